# Initial kernel scaffold; baseline (speedup 1.0000x reference)
import jax
import jax.numpy as jnp
from jax.experimental import pallas as pl


def kernel(x, edge_index, edge_type, batch, type_features, W_l, b_l, W_r, b_r, W_e, att, bias_conv, W_lin, b_lin):
    raise NotImplementedError("write your pallas kernel here")


# speedup vs baseline: 5.6914x; 5.6914x over previous
import functools

import jax
import jax.numpy as jnp
from jax import lax
from jax.experimental import pallas as pl
from jax.experimental.pallas import tpu as pltpu
from jax.experimental.pallas import tpu_sc as plsc

N = 10000
E = 160000
DIN = 128
HID = 64
HEADS = 8
DOUT = 64
NTYPES = 32
NG = 16
HD = HEADS * HID

NC, NS = 2, 16
NW = NC * NS
C = 64
NCHUNKS = E // C
KMAX = (NCHUNKS + NW - 1) // NW
C1 = 16
NCHUNKS1 = E // C1
KMAX1 = (NCHUNKS1 + NW - 1) // NW
C2 = 16
NCHUNKS2 = E // C2
KMAX2 = (NCHUNKS2 + NW - 1) // NW
NPAD = 10240
ROWS_PER_TILE = NPAD // NS

BN = 400
GN = N // BN
EB = E // GN

_f32 = jnp.float32


def _tc_a_body(x_ref, wl_ref, bl_ref, wr_ref, br_ref, tf_ref, we_ref,
               et_ref, xl_ref, xr_ref, tfe_ref, cnt_ref):
    xb = x_ref[...]
    xl_ref[...] = jnp.dot(xb, wl_ref[...],
                          preferred_element_type=_f32) + bl_ref[...]
    xr_ref[...] = jnp.dot(xb, wr_ref[...],
                          preferred_element_type=_f32) + br_ref[...]
    tfe_ref[...] = jnp.dot(tf_ref[...], we_ref[...],
                           preferred_element_type=_f32)
    et = et_ref[...].reshape(1, EB)
    ids = lax.broadcasted_iota(jnp.int32, (NTYPES, EB), 0)
    oh = (ids == jnp.broadcast_to(et, (NTYPES, EB))).astype(_f32)
    cnt_ref[...] = jnp.sum(oh, axis=1).reshape(1, 1, NTYPES)


def _tc_a(x, W_l, b_l, W_r, b_r, type_features, W_e, et3):
    return pl.pallas_call(
        _tc_a_body,
        grid=(GN,),
        in_specs=[
            pl.BlockSpec((BN, DIN), lambda i: (i, 0)),
            pl.BlockSpec((DIN, HD), lambda i: (0, 0)),
            pl.BlockSpec((1, HD), lambda i: (0, 0)),
            pl.BlockSpec((DIN, HD), lambda i: (0, 0)),
            pl.BlockSpec((1, HD), lambda i: (0, 0)),
            pl.BlockSpec((NTYPES, DIN), lambda i: (0, 0)),
            pl.BlockSpec((DIN, HD), lambda i: (0, 0)),
            pl.BlockSpec((1, 1, EB), lambda i: (i, 0, 0)),
        ],
        out_specs=[
            pl.BlockSpec((BN, HD), lambda i: (i, 0)),
            pl.BlockSpec((BN, HD), lambda i: (i, 0)),
            pl.BlockSpec((NTYPES, HD), lambda i: (0, 0)),
            pl.BlockSpec((1, 1, NTYPES), lambda i: (i, 0, 0)),
        ],
        out_shape=[
            jax.ShapeDtypeStruct((N, HD), _f32),
            jax.ShapeDtypeStruct((N, HD), _f32),
            jax.ShapeDtypeStruct((NTYPES, HD), _f32),
            jax.ShapeDtypeStruct((GN, 1, NTYPES), _f32),
        ],
    )(x, W_l, b_l, W_r, b_r, type_features, W_e, et3)


def _sc_p1_body(tab_hbm, idx_hbm, att_hbm,
                expa_hbm,
                idx_v, gbuf, attv, ab, sem1):
    cid = lax.axis_index("c")
    sid = lax.axis_index("s")
    wid = sid * NC + cid

    pltpu.sync_copy(att_hbm, attv)

    lanes = lax.iota(jnp.int32, 16)
    perms = [lanes ^ k for k in (8, 4, 2, 1)]

    def chunk_body(k, carry):
        c = wid + k * NW

        @pl.when(c < NCHUNKS1)
        def _():
            gbase = c * C1
            pltpu.sync_copy(idx_hbm.at[pl.ds(3 * gbase, 3 * C1)], idx_v)
            pltpu.async_copy(tab_hbm.at[idx_v], gbuf, sem1).wait()

            def edge_body(i, carry2):
                rowv = jnp.zeros((16,), _f32)
                for h in range(HEADS):
                    acc = jnp.zeros((16,), _f32)
                    for j4 in range(4):
                        j = h * 4 + j4
                        q = (gbuf[i, pl.ds(16 * j, 16)]
                             + gbuf[C1 + i, pl.ds(16 * j, 16)]
                             + gbuf[2 * C1 + i, pl.ds(16 * j, 16)])
                        m = jnp.maximum(q, 0.0) + 0.2 * jnp.minimum(q, 0.0)
                        acc = acc + m * attv[pl.ds(16 * j, 16)]
                    for p in perms:
                        acc = acc + jnp.take_along_axis(
                            acc, p, axis=0, mode="promise_in_bounds")
                    rowv = jnp.where(lanes == h, acc, rowv)
                ab[i, :] = jnp.where(lanes < HEADS, jnp.exp(rowv), 0.0)
                return carry2

            lax.fori_loop(0, C1, edge_body, 0)
            pltpu.sync_copy(ab, expa_hbm.at[pl.ds(gbase, C1)])

        return carry

    lax.fori_loop(0, KMAX1, chunk_body, 0)


def _sc_p1(TAB, IDX, att_flat):
    mesh = plsc.VectorSubcoreMesh(core_axis_name="c", subcore_axis_name="s",
                                  num_cores=NC, num_subcores=NS)
    fn = pl.kernel(
        _sc_p1_body,
        out_type=jax.ShapeDtypeStruct((E, 16), _f32),
        mesh=mesh,
        scratch_types=[
            pltpu.VMEM((3 * C1,), jnp.int32),
            pltpu.VMEM((3 * C1, HD), _f32),
            pltpu.VMEM((HD,), _f32),
            pltpu.VMEM((C1, 16), _f32),
            pltpu.SemaphoreType.DMA,
        ],
    )
    return fn(TAB, IDX, att_flat)


def _sc_p15_body(expa_hbm, den_hbm, dst_hbm, a_hbm,
                 dst_v, expab, denb, ab, sem1):
    cid = lax.axis_index("c")
    sid = lax.axis_index("s")
    wid = sid * NC + cid

    def chunk_body(k, carry):
        c = wid + k * NW

        @pl.when(c < NCHUNKS)
        def _():
            gbase = c * C
            pltpu.sync_copy(dst_hbm.at[pl.ds(gbase, C)], dst_v)
            pltpu.sync_copy(expa_hbm.at[pl.ds(gbase, C)], expab)
            pltpu.async_copy(den_hbm.at[dst_v], denb, sem1).wait()

            def row_body(i, carry2):
                ab[i, :] = expab[i, :] / (denb[i, pl.ds(0, 16)] + 1e-16)
                return carry2

            lax.fori_loop(0, C, row_body, 0)
            pltpu.sync_copy(ab, a_hbm.at[pl.ds(gbase, C)])

        return carry

    lax.fori_loop(0, KMAX, chunk_body, 0)


def _sc_p15(EXPA, DEN, dst):
    mesh = plsc.VectorSubcoreMesh(core_axis_name="c", subcore_axis_name="s",
                                  num_cores=NC, num_subcores=NS)
    fn = pl.kernel(
        _sc_p15_body,
        out_type=jax.ShapeDtypeStruct((E, 16), _f32),
        mesh=mesh,
        scratch_types=[
            pltpu.VMEM((C,), jnp.int32),
            pltpu.VMEM((C, 16), _f32),
            pltpu.VMEM((C, 128), _f32),
            pltpu.VMEM((C, 16), _f32),
            pltpu.SemaphoreType.DMA,
        ],
    )
    return fn(EXPA, DEN, dst)


def _tc_e0_body(expa_ref, dst_ref, den_ref, acc_ref):
    i = pl.program_id(0)
    d = dst_ref[...].reshape(1, BE)
    oh = (lax.broadcasted_iota(jnp.int32, (N, BE), 0)
          == jnp.broadcast_to(d, (N, BE))).astype(_f32)

    @pl.when(i == 0)
    def _():
        acc_ref[...] = jnp.zeros((N, 16), _f32)

    acc_ref[...] += jnp.dot(oh, expa_ref[...], preferred_element_type=_f32)

    @pl.when(i == GE - 1)
    def _():
        den_ref[...] = acc_ref[...]


def _tc_e0(EXPA, dst3):
    return pl.pallas_call(
        _tc_e0_body,
        grid=(GE,),
        in_specs=[
            pl.BlockSpec((BE, 16), lambda i: (i, 0)),
            pl.BlockSpec((1, 1, BE), lambda i: (i, 0, 0)),
        ],
        out_specs=pl.BlockSpec((N, 16), lambda i: (0, 0)),
        out_shape=jax.ShapeDtypeStruct((N, 16), _f32),
        scratch_shapes=[pltpu.VMEM((N, 16), _f32)],
    )(EXPA, dst3)


def _tc_b_body(cnt_ref, tfe_ref, xl_ref, xr_ref, denp_ref, asel_ref,
               den_ref, smsg_ref):
    cnt = jnp.sum(cnt_ref[...].reshape(GN, NTYPES), axis=0)
    efm = jnp.dot((cnt / E).reshape(1, NTYPES), tfe_ref[...],
                  preferred_element_type=_f32)
    xlb = xl_ref[...]
    q = xlb + xr_ref[...] + efm
    m = jnp.maximum(q, 0.0) + 0.2 * jnp.minimum(q, 0.0)
    alpha = jnp.dot(m, asel_ref[...], preferred_element_type=_f32)
    expa = jnp.exp(alpha)
    den8 = denp_ref[...][:, :HEADS] + expa
    den_ref[...] = jnp.concatenate(
        [den8, jnp.ones((BN, 128 - HEADS), _f32)], axis=1)
    a = expa / (den8 + 1e-16)
    acc = jnp.zeros((BN, HID), _f32)
    for h in range(HEADS):
        acc = acc + a[:, h:h + 1] * xlb[:, h * HID:(h + 1) * HID]
    smsg_ref[...] = acc * (1.0 / HEADS)


def _tc_b(CNT, TFE, XL, XR, DENP, Asel):
    return pl.pallas_call(
        _tc_b_body,
        grid=(GN,),
        in_specs=[
            pl.BlockSpec((GN, 1, NTYPES), lambda i: (0, 0, 0)),
            pl.BlockSpec((NTYPES, HD), lambda i: (0, 0)),
            pl.BlockSpec((BN, HD), lambda i: (i, 0)),
            pl.BlockSpec((BN, HD), lambda i: (i, 0)),
            pl.BlockSpec((BN, 16), lambda i: (i, 0)),
            pl.BlockSpec((HD, HEADS), lambda i: (0, 0)),
        ],
        out_specs=[
            pl.BlockSpec((BN, 128), lambda i: (i, 0)),
            pl.BlockSpec((BN, HID), lambda i: (i, 0)),
        ],
        out_shape=[
            jax.ShapeDtypeStruct((N, 128), _f32),
            jax.ShapeDtypeStruct((N, HID), _f32),
        ],
    )(CNT, TFE, XL, XR, DENP, Asel)


def _tc_d_body(a_ref, rep_ref, abc_ref):
    abc_ref[...] = jnp.dot(a_ref[...][:, :HEADS], rep_ref[...],
                           preferred_element_type=_f32)


def _tc_d(A, REP):
    BE = 2000
    return pl.pallas_call(
        _tc_d_body,
        grid=(E // BE,),
        in_specs=[
            pl.BlockSpec((BE, 16), lambda i: (i, 0)),
            pl.BlockSpec((HEADS, 128), lambda i: (0, 0)),
        ],
        out_specs=pl.BlockSpec((BE, 128), lambda i: (i, 0)),
        out_shape=jax.ShapeDtypeStruct((E, 128), _f32),
    )(A, REP)


def _sc_p2_body(tab_hbm, src_hbm, xls_hbm, src_v, xlb, sem1):
    cid = lax.axis_index("c")
    sid = lax.axis_index("s")
    wid = sid * NC + cid

    def chunk_body(k, carry):
        c = wid + k * NW

        @pl.when(c < NCHUNKS)
        def _():
            gbase = c * C
            pltpu.sync_copy(src_hbm.at[pl.ds(gbase, C)], src_v)
            pltpu.async_copy(tab_hbm.at[src_v], xlb, sem1).wait()
            pltpu.sync_copy(xlb, xls_hbm.at[pl.ds(gbase, C)])

        return carry

    lax.fori_loop(0, KMAX, chunk_body, 0)


def _sc_p2(TAB, src):
    mesh = plsc.VectorSubcoreMesh(core_axis_name="c", subcore_axis_name="s",
                                  num_cores=NC, num_subcores=NS)
    fn = pl.kernel(
        _sc_p2_body,
        out_type=jax.ShapeDtypeStruct((E, HD), _f32),
        mesh=mesh,
        scratch_types=[
            pltpu.VMEM((C,), jnp.int32),
            pltpu.VMEM((C, HD), _f32),
            pltpu.SemaphoreType.DMA,
        ],
    )
    return fn(TAB, src)


BE = 256
GE = E // BE


def _tc_e_body(xls_ref, a_ref, dst_ref, node_ref, acc_ref):
    i = pl.program_id(0)
    a = a_ref[...]
    xls = xls_ref[...]
    msg = jnp.zeros((BE, HID), _f32)
    for h in range(HEADS):
        msg = msg + a[:, h:h + 1] * xls[:, h * HID:(h + 1) * HID]
    msg = msg * (1.0 / HEADS)
    d = dst_ref[...].reshape(1, BE)
    oh = (lax.broadcasted_iota(jnp.int32, (N, BE), 0)
          == jnp.broadcast_to(d, (N, BE))).astype(_f32)

    @pl.when(i == 0)
    def _():
        acc_ref[...] = jnp.zeros((N, HID), _f32)

    acc_ref[...] += jnp.dot(oh, msg, preferred_element_type=_f32)

    @pl.when(i == GE - 1)
    def _():
        node_ref[...] = acc_ref[...]


def _tc_e(XLS, A, dst3):
    return pl.pallas_call(
        _tc_e_body,
        grid=(GE,),
        in_specs=[
            pl.BlockSpec((BE, HD), lambda i: (i, 0)),
            pl.BlockSpec((BE, 16), lambda i: (i, 0)),
            pl.BlockSpec((1, 1, BE), lambda i: (i, 0, 0)),
        ],
        out_specs=pl.BlockSpec((N, HID), lambda i: (0, 0)),
        out_shape=jax.ShapeDtypeStruct((N, HID), _f32),
        scratch_shapes=[pltpu.VMEM((N, HID), _f32)],
    )(XLS, A, dst3)


def _tc_c_body(node_ref, smsg_ref, batch_ref, bias_ref, wlin_ref, blin_ref,
               out_ref, acc_ref, cnt_ref):
    i = pl.program_id(0)
    node = node_ref[...] + smsg_ref[...] + bias_ref[...]
    h = jnp.tanh(node)
    b = batch_ref[...].reshape(1, BN)
    oh = (lax.broadcasted_iota(jnp.int32, (NG, BN), 0)
          == jnp.broadcast_to(b, (NG, BN))).astype(_f32)

    @pl.when(i == 0)
    def _():
        acc_ref[...] = jnp.zeros((NG, HID), _f32)
        cnt_ref[...] = jnp.zeros((NG, 128), _f32)

    acc_ref[...] += jnp.dot(oh, h, preferred_element_type=_f32)
    cnt_ref[...] += jnp.dot(oh, jnp.ones((BN, 128), _f32),
                            preferred_element_type=_f32)

    gmean = acc_ref[...] / jnp.maximum(cnt_ref[...][:, :HID], 1.0)
    out_ref[...] = jnp.tanh(
        jnp.dot(gmean, wlin_ref[...], preferred_element_type=_f32)
        + blin_ref[...])


def _tc_c(NODE, SMSG, batch3, bias2, W_lin, b_lin2):
    return pl.pallas_call(
        _tc_c_body,
        grid=(GN,),
        in_specs=[
            pl.BlockSpec((BN, HID), lambda i: (i, 0)),
            pl.BlockSpec((BN, HID), lambda i: (i, 0)),
            pl.BlockSpec((1, 1, BN), lambda i: (i, 0, 0)),
            pl.BlockSpec((1, HID), lambda i: (0, 0)),
            pl.BlockSpec((HID, DOUT), lambda i: (0, 0)),
            pl.BlockSpec((1, DOUT), lambda i: (0, 0)),
        ],
        out_specs=pl.BlockSpec((NG, DOUT), lambda i: (0, 0)),
        out_shape=jax.ShapeDtypeStruct((NG, DOUT), _f32),
        scratch_shapes=[
            pltpu.VMEM((NG, HID), _f32),
            pltpu.VMEM((NG, 128), _f32),
        ],
    )(NODE, SMSG, batch3, bias2, W_lin, b_lin2)


def kernel(x, edge_index, edge_type, batch, type_features, W_l, b_l,
           W_r, b_r, W_e, att, bias_conv, W_lin, b_lin):
    et3 = edge_type.reshape(GN, 1, EB)
    XL, XR, TFE, CNT = _tc_a(x, W_l, b_l.reshape(1, HD), W_r,
                             b_r.reshape(1, HD), type_features, W_e, et3)

    TAB = jnp.concatenate([XL, XR, TFE], axis=0)
    src = edge_index[0]
    dst = edge_index[1]
    IDX = jnp.concatenate([src.reshape(NCHUNKS1, C1),
                           dst.reshape(NCHUNKS1, C1) + N,
                           edge_type.reshape(NCHUNKS1, C1) + 2 * N],
                          axis=1).reshape(3 * E)

    att_flat = att.reshape(HD)
    EXPA = _sc_p1(TAB, IDX, att_flat)
    DEN16 = _tc_e0(EXPA, dst.reshape(GE, 1, BE))

    Asel = (att[:, :, None] * jnp.eye(HEADS, dtype=_f32)[:, None, :]
            ).reshape(HD, HEADS)
    DEN, SMSG = _tc_b(CNT, TFE, XL, XR, DEN16, Asel)

    A = _sc_p15(EXPA, DEN, dst)

    XLS = _sc_p2(TAB, src)
    NODE = _tc_e(XLS, A, dst.reshape(GE, 1, BE))

    return _tc_c(NODE, SMSG, batch.reshape(GN, 1, BN),
                 bias_conv.reshape(1, HID), W_lin, b_lin.reshape(1, DOUT))
